# baseline (speedup 1.0000x reference)
import functools

import jax
import jax.numpy as jnp
from jax import lax
from jax.experimental import pallas as pl
from jax.experimental.pallas import tpu as pltpu
from jax.experimental.pallas import tpu_sc as plsc

BATCH = 16384
EMBED_DIM = 64
NUM_ITEMS = 1000000

_info = plsc.get_sparse_core_info()
_NC, _NS = _info.num_cores, _info.num_subcores
_NW = _NC * _NS

_CHUNK_VALS = 512
_CHUNKS_PER_W = 62
_N_CHUNKS = _NW * _CHUNKS_PER_W
_SINK = BATCH
_OUT_ROWS = BATCH + 16
_LAST_FULL = (NUM_ITEMS // _CHUNK_VALS) * _CHUNK_VALS


@functools.partial(
    pl.kernel,
    mesh=plsc.VectorSubcoreMesh(core_axis_name="c", subcore_axis_name="s"),
    out_type=jax.ShapeDtypeStruct((_OUT_ROWS, 128), jnp.float32),
    compiler_params=pltpu.CompilerParams(needs_layout_passes=False),
    scratch_types=[
        pltpu.VMEM((BATCH,), jnp.int32),
        pltpu.VMEM((BATCH,), jnp.int32),
        pltpu.VMEM((2, EMBED_DIM, _CHUNK_VALS), jnp.float32),
        pltpu.VMEM((EMBED_DIM, NUM_ITEMS - _LAST_FULL), jnp.float32),
        pltpu.VMEM((EMBED_DIM, 128), jnp.float32),
        pltpu.VMEM((2, 16, 128), jnp.float32),
        pltpu.VMEM((80,), jnp.int32),
        pltpu.SemaphoreType.DMA,
    ],
)
def _gather_kernel(sidx_hbm, perm_hbm, bnd_hbm, tableT_hbm, out_hbm,
                   idx_l, perm_l, chunk3, last_buf, colstage,
                   rowstage, bnd_v, sem):
    wid = lax.axis_index("s") * _NC + lax.axis_index("c")
    pltpu.sync_copy(sidx_hbm, idx_l)
    pltpu.sync_copy(perm_hbm, perm_l)
    b0 = wid * _CHUNKS_PER_W
    pltpu.sync_copy(bnd_hbm.at[pl.ds(wid * 64, 64)], bnd_v.at[pl.ds(0, 64)])

    lane = lax.iota(jnp.int32, 16)

    def do_chunk(q, tg):
        hv = bnd_v[pl.ds(q, 16)]
        h0 = hv[0]
        h1 = hv[1]
        j = b0 + q
        vbase = j * _CHUNK_VALS
        par = q & 1
        partial = vbase + _CHUNK_VALS > NUM_ITEMS

        @pl.when((h1 > h0) & jnp.logical_not(partial))
        def _():
            pltpu.sync_copy(
                tableT_hbm.at[:, pl.ds(vbase, _CHUNK_VALS)], chunk3.at[par]
            )

        @pl.when((h1 > h0) & partial)
        def _():
            pltpu.sync_copy(
                tableT_hbm.at[:, pl.ds(_LAST_FULL, NUM_ITEMS - _LAST_FULL)],
                last_buf,
            )
            for r in range(EMBED_DIM):
                for m in range((NUM_ITEMS - _LAST_FULL) // 16):
                    chunk3[par, r, pl.ds(m * 16, 16)] = last_buf[
                        r, pl.ds(m * 16, 16)
                    ]

        h0vec = plsc.load_gather(bnd_v, [jnp.zeros((16,), jnp.int32) + q])
        h1vec = plsc.load_gather(bnd_v, [jnp.ones((16,), jnp.int32) + q])

        def make_group(gather_ref, width):
            def do_group(g, tg):
                rp = tg & 1
                pos = lane + g * 16
                valid = (pos >= h0vec) & (pos < h1vec)
                vvec = idx_l[pl.ds(g * 16, 16)]
                col = jnp.minimum(vvec & (_CHUNK_VALS - 1), width - 1)
                for r in range(EMBED_DIM):
                    rvec = jnp.full((16,), r, jnp.int32)
                    colstage[r, pl.ds(0, 16)] = plsc.load_gather(
                        gather_ref, [rvec, col]
                    )
                @pl.when(tg >= 2)
                def _():
                    pltpu.make_async_copy(
                        out_hbm.at[pl.ds(0, 16)], rowstage.at[rp], sem
                    ).wait()
                for k in range(16):
                    kvec = jnp.full((16,), k, jnp.int32)
                    for q4 in range(EMBED_DIM // 16):
                        rr = lane + q4 * 16
                        rowstage[rp, k, pl.ds(q4 * 16, 16)] = plsc.load_gather(
                            colstage, [rr, kvec]
                        )
                pvec = perm_l[pl.ds(g * 16, 16)]
                pm = jnp.where(valid, pvec, _SINK)
                pltpu.make_async_copy(
                    rowstage.at[rp], out_hbm.at[pm], sem
                ).start()
                return tg + 1

            return do_group

        g0 = h0 // 16
        g1 = (h1 + 15) // 16
        return lax.fori_loop(
            g0, g1, make_group(chunk3.at[par], _CHUNK_VALS), tg
        )

    tg = lax.fori_loop(0, _CHUNKS_PER_W, do_chunk, 0)
    @pl.when(tg >= 1)
    def _():
        pltpu.make_async_copy(
            out_hbm.at[pl.ds(0, 16)], rowstage.at[(tg - 1) & 1], sem
        ).wait()

    @pl.when(tg >= 2)
    def _():
        pltpu.make_async_copy(
            out_hbm.at[pl.ds(0, 16)], rowstage.at[tg & 1], sem
        ).wait()


@jax.jit
def kernel(item_indices, table):
    idx32 = item_indices.astype(jnp.int32)
    pos = lax.iota(jnp.int32, BATCH)
    sidx, perm = lax.sort_key_val(idx32, pos)
    i = jnp.arange(_NW * 64, dtype=jnp.int32)
    jj = (i // 64) * _CHUNKS_PER_W + jnp.minimum(i % 64, _CHUNKS_PER_W)
    bounds = jnp.searchsorted(sidx, jj * _CHUNK_VALS).astype(jnp.int32)
    out = _gather_kernel(sidx, perm, bounds, table.T)
    return out[:BATCH, :EMBED_DIM]
